# baseline (speedup 1.0000x reference)
import functools

import jax
import jax.numpy as jnp
from jax import lax
from jax.experimental import pallas as pl
from jax.experimental.pallas import tpu as pltpu
from jax.experimental.pallas import tpu_sc as plsc

IOU_THRESHOLD = 0.5
MAX_DET = 300
PAD_DET = 304
BATCH = 8
NUM_BOXES = 20000
L = 16
NBLK = PAD_DET // L
NVEC = NUM_BOXES // L
NO = 50
NI = 25
BAND = 1024
INT32_MAX = 2147483647
RA_KEY = 0
RA_IDX = NUM_BOXES
RB_KEY = 2 * NUM_BOXES
RB_IDX = 3 * NUM_BOXES


def _nms_body(scores_hbm, boxes_hbm, cls_hbm,
              keep_out, score_out, bx1_out, by1_out, bx2_out, by2_out,
              cls_out, cnt_out,
              big_v, idx_s, scores_v, hist,
              kx1, ky1, kx2, ky2, karea,
              st_keep, st_safe, st_score, st_cls, st_cnt,
              cnt_smem, c_smem, mx_smem):
    c = lax.axis_index("c")
    s = lax.axis_index("s")
    wid = s * 2 + c

    @pl.when(wid < BATCH)
    def _():
        b = wid
        lane = lax.iota(jnp.int32, L)
        lane0 = lane == 0
        ones = jnp.ones((L,), jnp.int32)
        zi = jnp.zeros((L,), jnp.int32)
        zf = jnp.zeros((L,), jnp.float32)
        neg1 = jnp.full((L,), -1, jnp.int32)
        m255 = jnp.full((L,), 255, jnp.int32)
        sh4 = jnp.full((L,), 4, jnp.int32)
        sh24 = jnp.full((L,), 24, jnp.int32)
        colfull = lane * jnp.int32(NVEC)

        pltpu.sync_copy(scores_hbm.at[b], scores_v)
        cnt_smem[0] = jnp.int32(0)
        c_smem[0] = jnp.int32(0)

        def round_body(r, rcarry):
            need = (r == 0) | ((cnt_smem[0] < MAX_DET)
                               & (c_smem[0] < NUM_BOXES))

            @pl.when(need)
            def _():
                target = jnp.where(r == 0, BAND, NUM_BOXES)

                def keys_body(v, cc):
                    p = v * L
                    sf = scores_v[pl.ds(p, L)]
                    bits = plsc.bitcast(sf, jnp.int32)
                    ku = jnp.where(bits < 0, ~bits,
                                   bits ^ jnp.int32(-2147483648))
                    kd = ~ku
                    big_v[pl.ds(RA_KEY + p, L)] = plsc.bitcast(
                        kd, jnp.float32)
                    iv = jnp.full((L,), p, jnp.int32) + lane
                    big_v[pl.ds(RA_IDX + p, L)] = plsc.bitcast(
                        iv, jnp.float32)
                    return cc

                lax.fori_loop(0, NVEC, keys_body, jnp.int32(0))

                def zero_body(i, cc):
                    hist[pl.ds(i * L, L)] = zi
                    return cc

                def pfx_body(i, carry_in):
                    vec = hist[pl.ds(i * L, L)]
                    inc = plsc.cumsum(vec)
                    hist[pl.ds(i * L, L)] = (
                        jnp.full((L,), carry_in) + inc - vec)
                    return carry_in + inc[L - 1]

                lax.fori_loop(0, 256, zero_body, jnp.int32(0))

                def hist8_body(v, cc):
                    kidx = colfull + v
                    kf = plsc.load_gather(big_v, [kidx])
                    k = plsc.bitcast(kf, jnp.int32)
                    d = lax.shift_right_logical(k, sh24) & m255
                    addr = lax.shift_left(d, sh4) | lane
                    plsc.addupdate_scatter(hist, [addr], ones)
                    return cc

                lax.fori_loop(0, NVEC, hist8_body, jnp.int32(0))

                def tot_body(d, carry):
                    cum, tt, csz = carry
                    vec = hist[pl.ds(d * L, L)]
                    ssum = lax.reduce_sum_p.bind(vec, axes=(0,))
                    cum2 = cum + ssum
                    newly = (tt < 0) & (cum2 >= target)
                    tt2 = jnp.where(newly, d, tt)
                    csz2 = jnp.where(newly, cum2, csz)
                    return (cum2, tt2, csz2)

                _, tdig, csz = lax.fori_loop(
                    0, 256, tot_body,
                    (jnp.int32(0), jnp.int32(-1), jnp.int32(0)))
                c_smem[0] = csz

                lax.fori_loop(0, 256, pfx_body, jnp.int32(0))

                tvec = jnp.full((L,), tdig)

                def compact_body(v, cc):
                    kidx = colfull + v
                    kf = plsc.load_gather(big_v, [kidx])
                    inf = plsc.load_gather(big_v, [kidx + NUM_BOXES])
                    k = plsc.bitcast(kf, jnp.int32)
                    d = lax.shift_right_logical(k, sh24) & m255
                    sel = d <= tvec
                    addr = lax.shift_left(d, sh4) | lane
                    ofs = plsc.load_gather(hist, [addr])
                    plsc.store_scatter(hist, [addr], ofs + 1, mask=sel)
                    plsc.store_scatter(big_v, [ofs + RB_KEY], kf, mask=sel)
                    plsc.store_scatter(big_v, [ofs + RB_IDX], inf, mask=sel)
                    return cc

                lax.fori_loop(0, NVEC, compact_body, jnp.int32(0))

                cpl = (csz + (L - 1)) // L
                cplv = jnp.full((L,), cpl)
                cv = jnp.full((L,), csz)
                colband = lane * cpl

                for p in range(4):
                    src_k = RB_KEY if p % 2 == 0 else RA_KEY
                    dst_k = RA_KEY if p % 2 == 0 else RB_KEY
                    shiftv = jnp.full((L,), p * 8, jnp.int32)

                    lax.fori_loop(0, 256, zero_body, jnp.int32(0))

                    def histb_outer(o, cc):
                        @pl.when(o * NI < cpl)
                        def _():
                            def histb_inner(vi, ci):
                                v = o * NI + vi
                                e = colband + v
                                m = (jnp.full((L,), v) < cplv) & (e < cv)
                                kf = plsc.load_gather(big_v, [e + src_k])
                                k = plsc.bitcast(kf, jnp.int32)
                                d = lax.shift_right_logical(k, shiftv) & m255
                                addr = lax.shift_left(d, sh4) | lane
                                plsc.addupdate_scatter(
                                    hist, [addr], ones, mask=m)
                                return ci

                            lax.fori_loop(0, NI, histb_inner, jnp.int32(0))

                        return cc

                    lax.fori_loop(0, NO, histb_outer, jnp.int32(0))
                    lax.fori_loop(0, 256, pfx_body, jnp.int32(0))

                    def scatb_outer(o, cc):
                        @pl.when(o * NI < cpl)
                        def _():
                            def scatb_inner(vi, ci):
                                v = o * NI + vi
                                e = colband + v
                                m = (jnp.full((L,), v) < cplv) & (e < cv)
                                kf = plsc.load_gather(big_v, [e + src_k])
                                inf = plsc.load_gather(
                                    big_v, [e + src_k + NUM_BOXES])
                                k = plsc.bitcast(kf, jnp.int32)
                                d = lax.shift_right_logical(k, shiftv) & m255
                                addr = lax.shift_left(d, sh4) | lane
                                ofs = plsc.load_gather(hist, [addr])
                                plsc.store_scatter(
                                    hist, [addr], ofs + 1, mask=m)
                                plsc.store_scatter(
                                    big_v, [ofs + dst_k], kf, mask=m)
                                plsc.store_scatter(
                                    big_v, [ofs + dst_k + NUM_BOXES], inf,
                                    mask=m)
                                return ci

                            lax.fori_loop(0, NI, scatb_inner, jnp.int32(0))

                        return cc

                    lax.fori_loop(0, NO, scatb_outer, jnp.int32(0))

                def save_body(v, cc):
                    p = v * L
                    f = big_v[pl.ds(RB_IDX + p, L)]
                    iv = plsc.bitcast(f, jnp.int32)
                    pos = jnp.full((L,), p, jnp.int32) + lane
                    idx_s[pl.ds(p, L)] = jnp.where(pos < cv, iv, 0)
                    return cc

                lax.fori_loop(0, NVEC, save_body, jnp.int32(0))

                pltpu.sync_copy(boxes_hbm.at[b], big_v)

                for k in range(NBLK):
                    sl = pl.ds(k * L, L)
                    kx1[sl] = zf
                    ky1[sl] = zf
                    kx2[sl] = zf
                    ky2[sl] = zf
                    karea[sl] = zf
                    st_keep[sl] = neg1
                    st_safe[sl] = zi
                cnt_smem[0] = jnp.int32(0)

                def scan_blk(blk, carry):
                    @pl.when((cnt_smem[0] < MAX_DET) & (blk * L < csz))
                    def _():
                        pos = blk * L
                        idx16 = idx_s[pl.ds(pos, L)]
                        i4 = idx16 * 4
                        cx1 = plsc.load_gather(big_v, [i4])
                        cy1 = plsc.load_gather(big_v, [i4 + 1])
                        cx2 = plsc.load_gather(big_v, [i4 + 2])
                        cy2 = plsc.load_gather(big_v, [i4 + 3])
                        careas = (cx2 - cx1) * (cy2 - cy1)

                        for j in range(L):
                            cnt = cnt_smem[0]
                            ax1 = jnp.full((L,), cx1[j])
                            ay1 = jnp.full((L,), cy1[j])
                            ax2 = jnp.full((L,), cx2[j])
                            ay2 = jnp.full((L,), cy2[j])
                            aar = jnp.full((L,), careas[j])
                            mx_smem[0] = jnp.float32(-1.0)

                            def iou_blk(k, acc):
                                @pl.when(k * L < cnt)
                                def _():
                                    sl = pl.ds(k * L, L)
                                    w = jnp.maximum(
                                        jnp.minimum(kx2[sl], ax2)
                                        - jnp.maximum(kx1[sl], ax1), 0.0)
                                    h = jnp.maximum(
                                        jnp.minimum(ky2[sl], ay2)
                                        - jnp.maximum(ky1[sl], ay1), 0.0)
                                    inter = w * h
                                    denom = karea[sl] + aar - inter
                                    met = inter - IOU_THRESHOLD * denom
                                    mx_smem[0] = jnp.maximum(
                                        mx_smem[0],
                                        lax.reduce_max_p.bind(
                                            met, axes=(0,)))

                                return acc

                            lax.fori_loop(0, NBLK, iou_blk, jnp.int32(0))
                            keep_j = ((cnt < MAX_DET)
                                      & (mx_smem[0] <= 0.0)
                                      & (pos + j < csz))
                            m = lane0 & jnp.full((L,), keep_j)
                            idxv = jnp.full((L,), cnt, jnp.int32)
                            plsc.store_scatter(kx1, [idxv], ax1, mask=m)
                            plsc.store_scatter(ky1, [idxv], ay1, mask=m)
                            plsc.store_scatter(kx2, [idxv], ax2, mask=m)
                            plsc.store_scatter(ky2, [idxv], ay2, mask=m)
                            plsc.store_scatter(karea, [idxv], aar, mask=m)
                            oidx = jnp.full((L,), idx16[j], jnp.int32)
                            plsc.store_scatter(st_keep, [idxv], oidx, mask=m)
                            plsc.store_scatter(st_safe, [idxv], oidx, mask=m)
                            cnt_smem[0] = cnt + keep_j.astype(jnp.int32)

                    return carry

                lax.fori_loop(0, NVEC, scan_blk, jnp.int32(0))

            return rcarry

        lax.fori_loop(0, 2, round_body, jnp.int32(0))

        cnt = cnt_smem[0]
        st_cnt[...] = jnp.full((L,), cnt, jnp.int32)

        pltpu.sync_copy(cls_hbm.at[b], idx_s)
        for k in range(NBLK):
            sl = pl.ds(k * L, L)
            kidx = st_safe[sl]
            raw = st_keep[sl]
            valid = raw >= 0
            sc = plsc.load_gather(scores_v, [kidx])
            cl = plsc.load_gather(idx_s, [kidx])
            st_score[sl] = jnp.where(valid, sc, 0.0)
            st_cls[sl] = jnp.where(valid, cl, INT32_MAX)

        pltpu.sync_copy(st_keep, keep_out.at[b])
        pltpu.sync_copy(st_score, score_out.at[b])
        pltpu.sync_copy(kx1, bx1_out.at[b])
        pltpu.sync_copy(ky1, by1_out.at[b])
        pltpu.sync_copy(kx2, bx2_out.at[b])
        pltpu.sync_copy(ky2, by2_out.at[b])
        pltpu.sync_copy(st_cls, cls_out.at[b])
        pltpu.sync_copy(st_cnt, cnt_out.at[b])


_sc_nms = functools.partial(
    pl.kernel,
    out_type=(
        jax.ShapeDtypeStruct((BATCH, PAD_DET), jnp.int32),
        jax.ShapeDtypeStruct((BATCH, PAD_DET), jnp.float32),
        jax.ShapeDtypeStruct((BATCH, PAD_DET), jnp.float32),
        jax.ShapeDtypeStruct((BATCH, PAD_DET), jnp.float32),
        jax.ShapeDtypeStruct((BATCH, PAD_DET), jnp.float32),
        jax.ShapeDtypeStruct((BATCH, PAD_DET), jnp.float32),
        jax.ShapeDtypeStruct((BATCH, PAD_DET), jnp.int32),
        jax.ShapeDtypeStruct((BATCH, L), jnp.int32),
    ),
    mesh=plsc.VectorSubcoreMesh(core_axis_name="c", subcore_axis_name="s"),
    compiler_params=pltpu.CompilerParams(needs_layout_passes=False),
    scratch_types=[
        pltpu.VMEM((4 * NUM_BOXES,), jnp.float32),
        pltpu.VMEM((NUM_BOXES,), jnp.int32),
        pltpu.VMEM((NUM_BOXES,), jnp.float32),
        pltpu.VMEM((4096,), jnp.int32),
        pltpu.VMEM((PAD_DET,), jnp.float32),
        pltpu.VMEM((PAD_DET,), jnp.float32),
        pltpu.VMEM((PAD_DET,), jnp.float32),
        pltpu.VMEM((PAD_DET,), jnp.float32),
        pltpu.VMEM((PAD_DET,), jnp.float32),
        pltpu.VMEM((PAD_DET,), jnp.int32),
        pltpu.VMEM((PAD_DET,), jnp.int32),
        pltpu.VMEM((PAD_DET,), jnp.float32),
        pltpu.VMEM((PAD_DET,), jnp.int32),
        pltpu.VMEM((L,), jnp.int32),
        pltpu.SMEM((1,), jnp.int32),
        pltpu.SMEM((1,), jnp.int32),
        pltpu.SMEM((1,), jnp.float32),
    ],
)(_nms_body)


def kernel(scores, boxes, classes):
    B, N = scores.shape
    boxes_flat = boxes.reshape(B, 4 * N)
    keep, osc, ox1, oy1, ox2, oy2, ocl, ocnt = _sc_nms(
        scores, boxes_flat, classes.astype(jnp.int32))
    out_boxes = jnp.stack([ox1, oy1, ox2, oy2], axis=-1)[:, :MAX_DET, :]
    return (
        keep[:, :MAX_DET],
        osc[:, :MAX_DET],
        out_boxes,
        ocl[:, :MAX_DET],
        ocnt[:, 0],
    )


# speedup vs baseline: 1.2358x; 1.2358x over previous
import functools

import jax
import jax.numpy as jnp
from jax import lax
from jax.experimental import pallas as pl
from jax.experimental.pallas import tpu as pltpu
from jax.experimental.pallas import tpu_sc as plsc

IOU_THRESHOLD = 0.5
MAX_DET = 300
PAD_DET = 304
BATCH = 8
NUM_BOXES = 20000
L = 16
NBLK = PAD_DET // L
NVEC = NUM_BOXES // L
NO = 50
NI = 25
BAND = 1024
INT32_MAX = 2147483647
RA_KEY = 0
RA_IDX = NUM_BOXES
RB_KEY = 2 * NUM_BOXES
RB_IDX = 3 * NUM_BOXES


def _nms_body(scores_hbm, boxes_hbm, cls_hbm,
              keep_out, score_out, bx1_out, by1_out, bx2_out, by2_out,
              cls_out, cnt_out,
              big_v, idx_s, scores_v, hist,
              kx1, ky1, kx2, ky2, karea,
              st_keep, st_safe, st_score, st_cls, st_cnt,
              cnt_smem, c_smem):
    c = lax.axis_index("c")
    s = lax.axis_index("s")
    wid = s * 2 + c

    @pl.when(wid < BATCH)
    def _():
        b = wid
        lane = lax.iota(jnp.int32, L)
        lane0 = lane == 0
        ones = jnp.ones((L,), jnp.int32)
        zi = jnp.zeros((L,), jnp.int32)
        zf = jnp.zeros((L,), jnp.float32)
        neg1 = jnp.full((L,), -1, jnp.int32)
        m255 = jnp.full((L,), 255, jnp.int32)
        sh4 = jnp.full((L,), 4, jnp.int32)
        sh24 = jnp.full((L,), 24, jnp.int32)
        colfull = lane * jnp.int32(NVEC)

        pltpu.sync_copy(scores_hbm.at[b], scores_v)
        cnt_smem[0] = jnp.int32(0)
        c_smem[0] = jnp.int32(0)

        def round_body(r, rcarry):
            need = (r == 0) | ((cnt_smem[0] < MAX_DET)
                               & (c_smem[0] < NUM_BOXES))

            @pl.when(need)
            def _():
                target = jnp.where(r == 0, BAND, NUM_BOXES)

                def keys_body(v, cc):
                    p = v * L
                    sf = scores_v[pl.ds(p, L)]
                    bits = plsc.bitcast(sf, jnp.int32)
                    ku = jnp.where(bits < 0, ~bits,
                                   bits ^ jnp.int32(-2147483648))
                    kd = ~ku
                    big_v[pl.ds(RA_KEY + p, L)] = plsc.bitcast(
                        kd, jnp.float32)
                    iv = jnp.full((L,), p, jnp.int32) + lane
                    big_v[pl.ds(RA_IDX + p, L)] = plsc.bitcast(
                        iv, jnp.float32)
                    return cc

                lax.fori_loop(0, NVEC, keys_body, jnp.int32(0))

                def zero_body(i, cc):
                    hist[pl.ds(i * L, L)] = zi
                    return cc

                def pfx_body(i, carry_in):
                    vec = hist[pl.ds(i * L, L)]
                    inc = plsc.cumsum(vec)
                    hist[pl.ds(i * L, L)] = (
                        jnp.full((L,), carry_in) + inc - vec)
                    return carry_in + inc[L - 1]

                lax.fori_loop(0, 256, zero_body, jnp.int32(0))

                def hist8_body(v, cc):
                    kidx = colfull + v
                    kf = plsc.load_gather(big_v, [kidx])
                    k = plsc.bitcast(kf, jnp.int32)
                    d = lax.shift_right_logical(k, sh24) & m255
                    addr = lax.shift_left(d, sh4) | lane
                    plsc.addupdate_scatter(hist, [addr], ones)
                    return cc

                lax.fori_loop(0, NVEC, hist8_body, jnp.int32(0))

                def tot_body(d, carry):
                    cum, tt, csz = carry
                    vec = hist[pl.ds(d * L, L)]
                    ssum = lax.reduce_sum_p.bind(vec, axes=(0,))
                    cum2 = cum + ssum
                    newly = (tt < 0) & (cum2 >= target)
                    tt2 = jnp.where(newly, d, tt)
                    csz2 = jnp.where(newly, cum2, csz)
                    return (cum2, tt2, csz2)

                _, tdig, csz = lax.fori_loop(
                    0, 256, tot_body,
                    (jnp.int32(0), jnp.int32(-1), jnp.int32(0)))
                c_smem[0] = csz

                lax.fori_loop(0, 256, pfx_body, jnp.int32(0))

                tvec = jnp.full((L,), tdig)

                def compact_body(v, cc):
                    kidx = colfull + v
                    kf = plsc.load_gather(big_v, [kidx])
                    inf = plsc.load_gather(big_v, [kidx + NUM_BOXES])
                    k = plsc.bitcast(kf, jnp.int32)
                    d = lax.shift_right_logical(k, sh24) & m255
                    sel = d <= tvec
                    addr = lax.shift_left(d, sh4) | lane
                    ofs = plsc.load_gather(hist, [addr])
                    plsc.store_scatter(hist, [addr], ofs + 1, mask=sel)
                    plsc.store_scatter(big_v, [ofs + RB_KEY], kf, mask=sel)
                    plsc.store_scatter(big_v, [ofs + RB_IDX], inf, mask=sel)
                    return cc

                lax.fori_loop(0, NVEC, compact_body, jnp.int32(0))

                cpl = (csz + (L - 1)) // L
                cplv = jnp.full((L,), cpl)
                cv = jnp.full((L,), csz)
                colband = lane * cpl

                for p in range(4):
                    src_k = RB_KEY if p % 2 == 0 else RA_KEY
                    dst_k = RA_KEY if p % 2 == 0 else RB_KEY
                    shiftv = jnp.full((L,), p * 8, jnp.int32)

                    lax.fori_loop(0, 256, zero_body, jnp.int32(0))

                    def histb_outer(o, cc):
                        @pl.when(o * NI < cpl)
                        def _():
                            def histb_inner(vi, ci):
                                v = o * NI + vi
                                e = colband + v
                                m = (jnp.full((L,), v) < cplv) & (e < cv)
                                kf = plsc.load_gather(big_v, [e + src_k])
                                k = plsc.bitcast(kf, jnp.int32)
                                d = lax.shift_right_logical(k, shiftv) & m255
                                addr = lax.shift_left(d, sh4) | lane
                                plsc.addupdate_scatter(
                                    hist, [addr], ones, mask=m)
                                return ci

                            lax.fori_loop(0, NI, histb_inner, jnp.int32(0))

                        return cc

                    lax.fori_loop(0, NO, histb_outer, jnp.int32(0))
                    lax.fori_loop(0, 256, pfx_body, jnp.int32(0))

                    def scatb_outer(o, cc):
                        @pl.when(o * NI < cpl)
                        def _():
                            def scatb_inner(vi, ci):
                                v = o * NI + vi
                                e = colband + v
                                m = (jnp.full((L,), v) < cplv) & (e < cv)
                                kf = plsc.load_gather(big_v, [e + src_k])
                                inf = plsc.load_gather(
                                    big_v, [e + src_k + NUM_BOXES])
                                k = plsc.bitcast(kf, jnp.int32)
                                d = lax.shift_right_logical(k, shiftv) & m255
                                addr = lax.shift_left(d, sh4) | lane
                                ofs = plsc.load_gather(hist, [addr])
                                plsc.store_scatter(
                                    hist, [addr], ofs + 1, mask=m)
                                plsc.store_scatter(
                                    big_v, [ofs + dst_k], kf, mask=m)
                                plsc.store_scatter(
                                    big_v, [ofs + dst_k + NUM_BOXES], inf,
                                    mask=m)
                                return ci

                            lax.fori_loop(0, NI, scatb_inner, jnp.int32(0))

                        return cc

                    lax.fori_loop(0, NO, scatb_outer, jnp.int32(0))

                def save_body(v, cc):
                    p = v * L
                    f = big_v[pl.ds(RB_IDX + p, L)]
                    iv = plsc.bitcast(f, jnp.int32)
                    pos = jnp.full((L,), p, jnp.int32) + lane
                    idx_s[pl.ds(p, L)] = jnp.where(pos < cv, iv, 0)
                    return cc

                lax.fori_loop(0, NVEC, save_body, jnp.int32(0))

                pltpu.sync_copy(boxes_hbm.at[b], big_v)

                for k in range(NBLK):
                    sl = pl.ds(k * L, L)
                    kx1[sl] = zf
                    ky1[sl] = zf
                    kx2[sl] = zf
                    ky2[sl] = zf
                    karea[sl] = zf
                    st_keep[sl] = neg1
                    st_safe[sl] = zi
                cnt_smem[0] = jnp.int32(0)

                def scan_blk(blk, carry):
                    @pl.when((cnt_smem[0] < MAX_DET) & (blk * L < csz))
                    def _():
                        pos = blk * L
                        idx16 = idx_s[pl.ds(pos, L)]
                        i4 = idx16 * 4
                        cx1 = plsc.load_gather(big_v, [i4])
                        cy1 = plsc.load_gather(big_v, [i4 + 1])
                        cx2 = plsc.load_gather(big_v, [i4 + 2])
                        cy2 = plsc.load_gather(big_v, [i4 + 3])
                        careas = (cx2 - cx1) * (cy2 - cy1)

                        for j in range(L):
                            cnt = cnt_smem[0]
                            ax1 = jnp.full((L,), cx1[j])
                            ay1 = jnp.full((L,), cy1[j])
                            ax2 = jnp.full((L,), cx2[j])
                            ay2 = jnp.full((L,), cy2[j])
                            aar = jnp.full((L,), careas[j])

                            def iou_blk(k, acc):
                                sl = pl.ds(k * L, L)
                                w = jnp.maximum(
                                    jnp.minimum(kx2[sl], ax2)
                                    - jnp.maximum(kx1[sl], ax1), 0.0)
                                h = jnp.maximum(
                                    jnp.minimum(ky2[sl], ay2)
                                    - jnp.maximum(ky1[sl], ay1), 0.0)
                                inter = w * h
                                denom = karea[sl] + aar - inter
                                return jnp.maximum(
                                    acc, inter - IOU_THRESHOLD * denom)

                            metric = lax.fori_loop(
                                0, NBLK, iou_blk,
                                jnp.full((L,), -1.0, jnp.float32))
                            mmax = lax.reduce_max_p.bind(metric, axes=(0,))
                            keep_j = ((cnt < MAX_DET)
                                      & (mmax <= 0.0)
                                      & (pos + j < csz))
                            m = lane0 & jnp.full((L,), keep_j)
                            idxv = jnp.full((L,), cnt, jnp.int32)
                            plsc.store_scatter(kx1, [idxv], ax1, mask=m)
                            plsc.store_scatter(ky1, [idxv], ay1, mask=m)
                            plsc.store_scatter(kx2, [idxv], ax2, mask=m)
                            plsc.store_scatter(ky2, [idxv], ay2, mask=m)
                            plsc.store_scatter(karea, [idxv], aar, mask=m)
                            oidx = jnp.full((L,), idx16[j], jnp.int32)
                            plsc.store_scatter(st_keep, [idxv], oidx, mask=m)
                            plsc.store_scatter(st_safe, [idxv], oidx, mask=m)
                            cnt_smem[0] = cnt + keep_j.astype(jnp.int32)

                    return carry

                lax.fori_loop(0, NVEC, scan_blk, jnp.int32(0))

            return rcarry

        lax.fori_loop(0, 2, round_body, jnp.int32(0))

        cnt = cnt_smem[0]
        st_cnt[...] = jnp.full((L,), cnt, jnp.int32)

        pltpu.sync_copy(cls_hbm.at[b], idx_s)
        for k in range(NBLK):
            sl = pl.ds(k * L, L)
            kidx = st_safe[sl]
            raw = st_keep[sl]
            valid = raw >= 0
            sc = plsc.load_gather(scores_v, [kidx])
            cl = plsc.load_gather(idx_s, [kidx])
            st_score[sl] = jnp.where(valid, sc, 0.0)
            st_cls[sl] = jnp.where(valid, cl, INT32_MAX)

        pltpu.sync_copy(st_keep, keep_out.at[b])
        pltpu.sync_copy(st_score, score_out.at[b])
        pltpu.sync_copy(kx1, bx1_out.at[b])
        pltpu.sync_copy(ky1, by1_out.at[b])
        pltpu.sync_copy(kx2, bx2_out.at[b])
        pltpu.sync_copy(ky2, by2_out.at[b])
        pltpu.sync_copy(st_cls, cls_out.at[b])
        pltpu.sync_copy(st_cnt, cnt_out.at[b])


_sc_nms = functools.partial(
    pl.kernel,
    out_type=(
        jax.ShapeDtypeStruct((BATCH, PAD_DET), jnp.int32),
        jax.ShapeDtypeStruct((BATCH, PAD_DET), jnp.float32),
        jax.ShapeDtypeStruct((BATCH, PAD_DET), jnp.float32),
        jax.ShapeDtypeStruct((BATCH, PAD_DET), jnp.float32),
        jax.ShapeDtypeStruct((BATCH, PAD_DET), jnp.float32),
        jax.ShapeDtypeStruct((BATCH, PAD_DET), jnp.float32),
        jax.ShapeDtypeStruct((BATCH, PAD_DET), jnp.int32),
        jax.ShapeDtypeStruct((BATCH, L), jnp.int32),
    ),
    mesh=plsc.VectorSubcoreMesh(core_axis_name="c", subcore_axis_name="s"),
    compiler_params=pltpu.CompilerParams(needs_layout_passes=False),
    scratch_types=[
        pltpu.VMEM((4 * NUM_BOXES,), jnp.float32),
        pltpu.VMEM((NUM_BOXES,), jnp.int32),
        pltpu.VMEM((NUM_BOXES,), jnp.float32),
        pltpu.VMEM((4096,), jnp.int32),
        pltpu.VMEM((PAD_DET,), jnp.float32),
        pltpu.VMEM((PAD_DET,), jnp.float32),
        pltpu.VMEM((PAD_DET,), jnp.float32),
        pltpu.VMEM((PAD_DET,), jnp.float32),
        pltpu.VMEM((PAD_DET,), jnp.float32),
        pltpu.VMEM((PAD_DET,), jnp.int32),
        pltpu.VMEM((PAD_DET,), jnp.int32),
        pltpu.VMEM((PAD_DET,), jnp.float32),
        pltpu.VMEM((PAD_DET,), jnp.int32),
        pltpu.VMEM((L,), jnp.int32),
        pltpu.SMEM((1,), jnp.int32),
        pltpu.SMEM((1,), jnp.int32),
    ],
)(_nms_body)


def kernel(scores, boxes, classes):
    B, N = scores.shape
    boxes_flat = boxes.reshape(B, 4 * N)
    keep, osc, ox1, oy1, ox2, oy2, ocl, ocnt = _sc_nms(
        scores, boxes_flat, classes.astype(jnp.int32))
    out_boxes = jnp.stack([ox1, oy1, ox2, oy2], axis=-1)[:, :MAX_DET, :]
    return (
        keep[:, :MAX_DET],
        osc[:, :MAX_DET],
        out_boxes,
        ocl[:, :MAX_DET],
        ocnt[:, 0],
    )


# speedup vs baseline: 1.2893x; 1.0433x over previous
import functools

import jax
import jax.numpy as jnp
from jax import lax
from jax.experimental import pallas as pl
from jax.experimental.pallas import tpu as pltpu
from jax.experimental.pallas import tpu_sc as plsc

IOU_THRESHOLD = 0.5
MAX_DET = 300
PAD_DET = 304
BATCH = 8
NUM_BOXES = 20000
L = 16
NBLK = PAD_DET // L
NVEC = NUM_BOXES // L
NO = 50
NI = 25
BAND = 1024
INT32_MAX = 2147483647
RB_KEY = 0
RB_IDX = NUM_BOXES
RA_KEY = 2 * NUM_BOXES
RA_IDX = 3 * NUM_BOXES


def _nms_body(scores_hbm, boxes_hbm, cls_hbm,
              keep_out, score_out, bx1_out, by1_out, bx2_out, by2_out,
              cls_out, cnt_out,
              big_v, idx_s, scores_v, hist,
              kx1, ky1, kx2, ky2, karea,
              st_keep, st_safe, st_score, st_cls, st_cnt,
              cnt_smem, c_smem):
    c = lax.axis_index("c")
    s = lax.axis_index("s")
    wid = s * 2 + c

    @pl.when(wid < BATCH)
    def _():
        b = wid
        lane = lax.iota(jnp.int32, L)
        lane0 = lane == 0
        ones = jnp.ones((L,), jnp.int32)
        zi = jnp.zeros((L,), jnp.int32)
        zf = jnp.zeros((L,), jnp.float32)
        neg1 = jnp.full((L,), -1, jnp.int32)
        m255 = jnp.full((L,), 255, jnp.int32)
        sh4 = jnp.full((L,), 4, jnp.int32)
        sh24 = jnp.full((L,), 24, jnp.int32)
        colfull = lane * jnp.int32(NVEC)

        pltpu.sync_copy(scores_hbm.at[b], scores_v)
        cnt_smem[0] = jnp.int32(0)
        c_smem[0] = jnp.int32(0)

        def round_body(r, rcarry):
            need = (r == 0) | ((cnt_smem[0] < MAX_DET)
                               & (c_smem[0] < NUM_BOXES))

            @pl.when(need)
            def _():
                target = jnp.where(r == 0, BAND, NUM_BOXES)

                def keys_body(v, cc):
                    p = v * L
                    sf = scores_v[pl.ds(p, L)]
                    bits = plsc.bitcast(sf, jnp.int32)
                    ku = jnp.where(bits < 0, ~bits,
                                   bits ^ jnp.int32(-2147483648))
                    kd = ~ku
                    big_v[pl.ds(RA_KEY + p, L)] = plsc.bitcast(
                        kd, jnp.float32)
                    iv = jnp.full((L,), p, jnp.int32) + lane
                    big_v[pl.ds(RA_IDX + p, L)] = plsc.bitcast(
                        iv, jnp.float32)
                    return cc

                lax.fori_loop(0, NVEC, keys_body, jnp.int32(0))

                def zero_body(i, cc):
                    hist[pl.ds(i * L, L)] = zi
                    return cc

                def pfx_body(i, carry_in):
                    vec = hist[pl.ds(i * L, L)]
                    inc = plsc.cumsum(vec)
                    hist[pl.ds(i * L, L)] = (
                        jnp.full((L,), carry_in) + inc - vec)
                    return carry_in + inc[L - 1]

                lax.fori_loop(0, 256, zero_body, jnp.int32(0))

                def hist8_body(v, cc):
                    kf = big_v[pl.ds(RA_KEY + v * L, L)]
                    k = plsc.bitcast(kf, jnp.int32)
                    d = lax.shift_right_logical(k, sh24) & m255
                    addr = lax.shift_left(d, sh4) | lane
                    plsc.addupdate_scatter(hist, [addr], ones)
                    return cc

                lax.fori_loop(0, NVEC, hist8_body, jnp.int32(0))

                def tot_body(d, carry):
                    cum, tt, csz = carry
                    vec = hist[pl.ds(d * L, L)]
                    ssum = lax.reduce_sum_p.bind(vec, axes=(0,))
                    cum2 = cum + ssum
                    newly = (tt < 0) & (cum2 >= target)
                    tt2 = jnp.where(newly, d, tt)
                    csz2 = jnp.where(newly, cum2, csz)
                    return (cum2, tt2, csz2)

                _, tdig, csz = lax.fori_loop(
                    0, 256, tot_body,
                    (jnp.int32(0), jnp.int32(-1), jnp.int32(0)))
                c_smem[0] = csz

                tvec = jnp.full((L,), tdig)

                def compact_body(v, off):
                    p = v * L
                    kf = big_v[pl.ds(RA_KEY + p, L)]
                    inf = big_v[pl.ds(RA_IDX + p, L)]
                    k = plsc.bitcast(kf, jnp.int32)
                    d = lax.shift_right_logical(k, sh24) & m255
                    sel = d <= tvec
                    plsc.store_compressed(
                        big_v.at[pl.ds(RB_KEY + off, L)], kf, mask=sel)
                    plsc.store_compressed(
                        big_v.at[pl.ds(RB_IDX + off, L)], inf, mask=sel)
                    pc = plsc.all_reduce_population_count(sel)
                    return off + pc[0]

                lax.fori_loop(0, NVEC, compact_body, jnp.int32(0))

                cpl = (csz + (L - 1)) // L
                cplv = jnp.full((L,), cpl)
                cv = jnp.full((L,), csz)
                colband = lane * cpl

                for p in range(4):
                    src_k = RB_KEY if p % 2 == 0 else RA_KEY
                    dst_k = RA_KEY if p % 2 == 0 else RB_KEY
                    shiftv = jnp.full((L,), p * 8, jnp.int32)

                    lax.fori_loop(0, 256, zero_body, jnp.int32(0))

                    def histb_outer(o, cc):
                        @pl.when(o * NI < cpl)
                        def _():
                            def histb_inner(vi, ci):
                                v = o * NI + vi
                                e = colband + v
                                m = (jnp.full((L,), v) < cplv) & (e < cv)
                                kf = plsc.load_gather(big_v, [e + src_k])
                                k = plsc.bitcast(kf, jnp.int32)
                                d = lax.shift_right_logical(k, shiftv) & m255
                                addr = lax.shift_left(d, sh4) | lane
                                plsc.addupdate_scatter(
                                    hist, [addr], ones, mask=m)
                                return ci

                            lax.fori_loop(0, NI, histb_inner, jnp.int32(0))

                        return cc

                    lax.fori_loop(0, NO, histb_outer, jnp.int32(0))
                    lax.fori_loop(0, 256, pfx_body, jnp.int32(0))

                    def scatb_outer(o, cc):
                        @pl.when(o * NI < cpl)
                        def _():
                            def scatb_inner(vi, ci):
                                v = o * NI + vi
                                e = colband + v
                                m = (jnp.full((L,), v) < cplv) & (e < cv)
                                kf = plsc.load_gather(big_v, [e + src_k])
                                inf = plsc.load_gather(
                                    big_v, [e + src_k + NUM_BOXES])
                                k = plsc.bitcast(kf, jnp.int32)
                                d = lax.shift_right_logical(k, shiftv) & m255
                                addr = lax.shift_left(d, sh4) | lane
                                ofs = plsc.load_gather(hist, [addr])
                                plsc.store_scatter(
                                    hist, [addr], ofs + 1, mask=m)
                                plsc.store_scatter(
                                    big_v, [ofs + dst_k], kf, mask=m)
                                plsc.store_scatter(
                                    big_v, [ofs + dst_k + NUM_BOXES], inf,
                                    mask=m)
                                return ci

                            lax.fori_loop(0, NI, scatb_inner, jnp.int32(0))

                        return cc

                    lax.fori_loop(0, NO, scatb_outer, jnp.int32(0))

                def save_body(v, cc):
                    p = v * L
                    f = big_v[pl.ds(RB_IDX + p, L)]
                    iv = plsc.bitcast(f, jnp.int32)
                    pos = jnp.full((L,), p, jnp.int32) + lane
                    idx_s[pl.ds(p, L)] = jnp.where(pos < cv, iv, 0)
                    return cc

                lax.fori_loop(0, NVEC, save_body, jnp.int32(0))

                pltpu.sync_copy(boxes_hbm.at[b], big_v)

                for k in range(NBLK):
                    sl = pl.ds(k * L, L)
                    kx1[sl] = zf
                    ky1[sl] = zf
                    kx2[sl] = zf
                    ky2[sl] = zf
                    karea[sl] = zf
                    st_keep[sl] = neg1
                    st_safe[sl] = zi
                cnt_smem[0] = jnp.int32(0)

                def scan_blk(blk, carry):
                    @pl.when((cnt_smem[0] < MAX_DET) & (blk * L < csz))
                    def _():
                        pos = blk * L
                        idx16 = idx_s[pl.ds(pos, L)]
                        i4 = idx16 * 4
                        cx1 = plsc.load_gather(big_v, [i4])
                        cy1 = plsc.load_gather(big_v, [i4 + 1])
                        cx2 = plsc.load_gather(big_v, [i4 + 2])
                        cy2 = plsc.load_gather(big_v, [i4 + 3])
                        careas = (cx2 - cx1) * (cy2 - cy1)

                        for j in range(L):
                            cnt = cnt_smem[0]
                            ax1 = jnp.full((L,), cx1[j])
                            ay1 = jnp.full((L,), cy1[j])
                            ax2 = jnp.full((L,), cx2[j])
                            ay2 = jnp.full((L,), cy2[j])
                            aar = jnp.full((L,), careas[j])

                            def iou_blk(k, acc):
                                sl = pl.ds(k * L, L)
                                w = jnp.maximum(
                                    jnp.minimum(kx2[sl], ax2)
                                    - jnp.maximum(kx1[sl], ax1), 0.0)
                                h = jnp.maximum(
                                    jnp.minimum(ky2[sl], ay2)
                                    - jnp.maximum(ky1[sl], ay1), 0.0)
                                inter = w * h
                                denom = karea[sl] + aar - inter
                                return jnp.maximum(
                                    acc, inter - IOU_THRESHOLD * denom)

                            metric = lax.fori_loop(
                                0, NBLK, iou_blk,
                                jnp.full((L,), -1.0, jnp.float32))
                            mmax = lax.reduce_max_p.bind(metric, axes=(0,))
                            keep_j = ((cnt < MAX_DET)
                                      & (mmax <= 0.0)
                                      & (pos + j < csz))
                            m = lane0 & jnp.full((L,), keep_j)
                            idxv = jnp.full((L,), cnt, jnp.int32)
                            plsc.store_scatter(kx1, [idxv], ax1, mask=m)
                            plsc.store_scatter(ky1, [idxv], ay1, mask=m)
                            plsc.store_scatter(kx2, [idxv], ax2, mask=m)
                            plsc.store_scatter(ky2, [idxv], ay2, mask=m)
                            plsc.store_scatter(karea, [idxv], aar, mask=m)
                            oidx = jnp.full((L,), idx16[j], jnp.int32)
                            plsc.store_scatter(st_keep, [idxv], oidx, mask=m)
                            plsc.store_scatter(st_safe, [idxv], oidx, mask=m)
                            cnt_smem[0] = cnt + keep_j.astype(jnp.int32)

                    return carry

                lax.fori_loop(0, NVEC, scan_blk, jnp.int32(0))

            return rcarry

        lax.fori_loop(0, 2, round_body, jnp.int32(0))

        cnt = cnt_smem[0]
        st_cnt[...] = jnp.full((L,), cnt, jnp.int32)

        pltpu.sync_copy(cls_hbm.at[b], idx_s)
        for k in range(NBLK):
            sl = pl.ds(k * L, L)
            kidx = st_safe[sl]
            raw = st_keep[sl]
            valid = raw >= 0
            sc = plsc.load_gather(scores_v, [kidx])
            cl = plsc.load_gather(idx_s, [kidx])
            st_score[sl] = jnp.where(valid, sc, 0.0)
            st_cls[sl] = jnp.where(valid, cl, INT32_MAX)

        pltpu.sync_copy(st_keep, keep_out.at[b])
        pltpu.sync_copy(st_score, score_out.at[b])
        pltpu.sync_copy(kx1, bx1_out.at[b])
        pltpu.sync_copy(ky1, by1_out.at[b])
        pltpu.sync_copy(kx2, bx2_out.at[b])
        pltpu.sync_copy(ky2, by2_out.at[b])
        pltpu.sync_copy(st_cls, cls_out.at[b])
        pltpu.sync_copy(st_cnt, cnt_out.at[b])


_sc_nms = functools.partial(
    pl.kernel,
    out_type=(
        jax.ShapeDtypeStruct((BATCH, PAD_DET), jnp.int32),
        jax.ShapeDtypeStruct((BATCH, PAD_DET), jnp.float32),
        jax.ShapeDtypeStruct((BATCH, PAD_DET), jnp.float32),
        jax.ShapeDtypeStruct((BATCH, PAD_DET), jnp.float32),
        jax.ShapeDtypeStruct((BATCH, PAD_DET), jnp.float32),
        jax.ShapeDtypeStruct((BATCH, PAD_DET), jnp.float32),
        jax.ShapeDtypeStruct((BATCH, PAD_DET), jnp.int32),
        jax.ShapeDtypeStruct((BATCH, L), jnp.int32),
    ),
    mesh=plsc.VectorSubcoreMesh(core_axis_name="c", subcore_axis_name="s"),
    compiler_params=pltpu.CompilerParams(needs_layout_passes=False),
    scratch_types=[
        pltpu.VMEM((4 * NUM_BOXES,), jnp.float32),
        pltpu.VMEM((NUM_BOXES,), jnp.int32),
        pltpu.VMEM((NUM_BOXES,), jnp.float32),
        pltpu.VMEM((4096,), jnp.int32),
        pltpu.VMEM((PAD_DET,), jnp.float32),
        pltpu.VMEM((PAD_DET,), jnp.float32),
        pltpu.VMEM((PAD_DET,), jnp.float32),
        pltpu.VMEM((PAD_DET,), jnp.float32),
        pltpu.VMEM((PAD_DET,), jnp.float32),
        pltpu.VMEM((PAD_DET,), jnp.int32),
        pltpu.VMEM((PAD_DET,), jnp.int32),
        pltpu.VMEM((PAD_DET,), jnp.float32),
        pltpu.VMEM((PAD_DET,), jnp.int32),
        pltpu.VMEM((L,), jnp.int32),
        pltpu.SMEM((1,), jnp.int32),
        pltpu.SMEM((1,), jnp.int32),
    ],
)(_nms_body)


def kernel(scores, boxes, classes):
    B, N = scores.shape
    boxes_flat = boxes.reshape(B, 4 * N)
    keep, osc, ox1, oy1, ox2, oy2, ocl, ocnt = _sc_nms(
        scores, boxes_flat, classes.astype(jnp.int32))
    out_boxes = jnp.stack([ox1, oy1, ox2, oy2], axis=-1)[:, :MAX_DET, :]
    return (
        keep[:, :MAX_DET],
        osc[:, :MAX_DET],
        out_boxes,
        ocl[:, :MAX_DET],
        ocnt[:, 0],
    )


# speedup vs baseline: 1.3287x; 1.0306x over previous
import functools

import jax
import jax.numpy as jnp
from jax import lax
from jax.experimental import pallas as pl
from jax.experimental.pallas import tpu as pltpu
from jax.experimental.pallas import tpu_sc as plsc

IOU_THRESHOLD = 0.5
MAX_DET = 300
PAD_DET = 304
BATCH = 8
NUM_BOXES = 20000
L = 16
NBLK = PAD_DET // L
NVEC = NUM_BOXES // L
NO = 50
NI = 25
BAND = 1024
INT32_MAX = 2147483647
RB_KEY = 0
RB_IDX = NUM_BOXES
RA_KEY = 2 * NUM_BOXES
RA_IDX = 3 * NUM_BOXES


def _nms_body(scores_hbm, boxes_hbm, cls_hbm,
              keep_out, score_out, bx1_out, by1_out, bx2_out, by2_out,
              cls_out, cnt_out,
              big_v, idx_s, scores_v, hist,
              kx1, ky1, kx2, ky2, karea,
              st_keep, st_safe, st_score, st_cls, st_cnt,
              cnt_smem, c_smem):
    c = lax.axis_index("c")
    s = lax.axis_index("s")
    wid = s * 2 + c

    @pl.when(wid < BATCH)
    def _():
        b = wid
        lane = lax.iota(jnp.int32, L)
        lane0 = lane == 0
        ones = jnp.ones((L,), jnp.int32)
        zi = jnp.zeros((L,), jnp.int32)
        zf = jnp.zeros((L,), jnp.float32)
        neg1 = jnp.full((L,), -1, jnp.int32)
        m255 = jnp.full((L,), 255, jnp.int32)
        sh4 = jnp.full((L,), 4, jnp.int32)
        sh24 = jnp.full((L,), 24, jnp.int32)
        colfull = lane * jnp.int32(NVEC)

        pltpu.sync_copy(scores_hbm.at[b], scores_v)
        cnt_smem[0] = jnp.int32(0)
        c_smem[0] = jnp.int32(0)

        def round_body(r, rcarry):
            need = (r == 0) | ((cnt_smem[0] < MAX_DET)
                               & (c_smem[0] < NUM_BOXES))

            @pl.when(need)
            def _():
                target = jnp.where(r == 0, BAND, NUM_BOXES)

                def mkkey(sf):
                    bits = plsc.bitcast(sf, jnp.int32)
                    ku = jnp.where(bits < 0, ~bits,
                                   bits ^ jnp.int32(-2147483648))
                    return ~ku

                def zero_body(i, cc):
                    hist[pl.ds(i * L, L)] = zi
                    return cc

                def pfx_body(i, carry_in):
                    vec = hist[pl.ds(i * L, L)]
                    inc = plsc.cumsum(vec)
                    hist[pl.ds(i * L, L)] = (
                        jnp.full((L,), carry_in) + inc - vec)
                    return carry_in + inc[L - 1]

                lax.fori_loop(0, 256, zero_body, jnp.int32(0))

                def hist8_body(v, cc):
                    k = mkkey(scores_v[pl.ds(v * L, L)])
                    d = lax.shift_right_logical(k, sh24) & m255
                    addr = lax.shift_left(d, sh4) | lane
                    plsc.addupdate_scatter(hist, [addr], ones)
                    return cc

                lax.fori_loop(0, NVEC, hist8_body, jnp.int32(0))

                def tot_body(d, carry):
                    cum, tt, csz = carry
                    vec = hist[pl.ds(d * L, L)]
                    ssum = lax.reduce_sum_p.bind(vec, axes=(0,))
                    cum2 = cum + ssum
                    newly = (tt < 0) & (cum2 >= target)
                    tt2 = jnp.where(newly, d, tt)
                    csz2 = jnp.where(newly, cum2, csz)
                    return (cum2, tt2, csz2)

                _, tdig, csz = lax.fori_loop(
                    0, 256, tot_body,
                    (jnp.int32(0), jnp.int32(-1), jnp.int32(0)))
                c_smem[0] = csz

                tvec = jnp.full((L,), tdig)

                def compact_body(v, off):
                    p = v * L
                    k = mkkey(scores_v[pl.ds(p, L)])
                    iv = jnp.full((L,), p, jnp.int32) + lane
                    d = lax.shift_right_logical(k, sh24) & m255
                    sel = d <= tvec
                    plsc.store_compressed(
                        big_v.at[pl.ds(RB_KEY + off, L)],
                        plsc.bitcast(k, jnp.float32), mask=sel)
                    plsc.store_compressed(
                        big_v.at[pl.ds(RB_IDX + off, L)],
                        plsc.bitcast(iv, jnp.float32), mask=sel)
                    pc = plsc.all_reduce_population_count(sel)
                    return off + pc[0]

                lax.fori_loop(0, NVEC, compact_body, jnp.int32(0))

                cpl = (csz + (L - 1)) // L
                cplv = jnp.full((L,), cpl)
                cv = jnp.full((L,), csz)
                colband = lane * cpl

                for p in range(4):
                    src_k = RB_KEY if p % 2 == 0 else RA_KEY
                    dst_k = RA_KEY if p % 2 == 0 else RB_KEY
                    shiftv = jnp.full((L,), p * 8, jnp.int32)

                    lax.fori_loop(0, 256, zero_body, jnp.int32(0))

                    def histb_outer(o, cc):
                        @pl.when(o * NI < cpl)
                        def _():
                            def histb_inner(vi, ci):
                                v = o * NI + vi
                                e = colband + v
                                m = (jnp.full((L,), v) < cplv) & (e < cv)
                                kf = plsc.load_gather(big_v, [e + src_k])
                                k = plsc.bitcast(kf, jnp.int32)
                                d = lax.shift_right_logical(k, shiftv) & m255
                                addr = lax.shift_left(d, sh4) | lane
                                plsc.addupdate_scatter(
                                    hist, [addr], ones, mask=m)
                                return ci

                            lax.fori_loop(0, NI, histb_inner, jnp.int32(0))

                        return cc

                    lax.fori_loop(0, NO, histb_outer, jnp.int32(0))
                    lax.fori_loop(0, 256, pfx_body, jnp.int32(0))

                    def scatb_outer(o, cc):
                        @pl.when(o * NI < cpl)
                        def _():
                            def scatb_inner(vi, ci):
                                v = o * NI + vi
                                e = colband + v
                                m = (jnp.full((L,), v) < cplv) & (e < cv)
                                kf = plsc.load_gather(big_v, [e + src_k])
                                inf = plsc.load_gather(
                                    big_v, [e + src_k + NUM_BOXES])
                                k = plsc.bitcast(kf, jnp.int32)
                                d = lax.shift_right_logical(k, shiftv) & m255
                                addr = lax.shift_left(d, sh4) | lane
                                ofs = plsc.load_gather(hist, [addr])
                                plsc.store_scatter(
                                    hist, [addr], ofs + 1, mask=m)
                                plsc.store_scatter(
                                    big_v, [ofs + dst_k], kf, mask=m)
                                plsc.store_scatter(
                                    big_v, [ofs + dst_k + NUM_BOXES], inf,
                                    mask=m)
                                return ci

                            lax.fori_loop(0, NI, scatb_inner, jnp.int32(0))

                        return cc

                    lax.fori_loop(0, NO, scatb_outer, jnp.int32(0))

                def save_outer(o, cc):
                    @pl.when(o * (NI * L) < csz + L)
                    def _():
                        def save_body(vi, ci):
                            p = (o * NI + vi) * L
                            f = big_v[pl.ds(RB_IDX + p, L)]
                            iv = plsc.bitcast(f, jnp.int32)
                            pos = jnp.full((L,), p, jnp.int32) + lane
                            idx_s[pl.ds(p, L)] = jnp.where(pos < cv, iv, 0)
                            return ci

                        lax.fori_loop(0, NI, save_body, jnp.int32(0))

                    return cc

                lax.fori_loop(0, NO, save_outer, jnp.int32(0))

                pltpu.sync_copy(boxes_hbm.at[b], big_v)

                for k in range(NBLK):
                    sl = pl.ds(k * L, L)
                    kx1[sl] = zf
                    ky1[sl] = zf
                    kx2[sl] = zf
                    ky2[sl] = zf
                    karea[sl] = zf
                    st_keep[sl] = neg1
                    st_safe[sl] = zi
                cnt_smem[0] = jnp.int32(0)

                def scan_blk(blk, carry):
                    @pl.when((cnt_smem[0] < MAX_DET) & (blk * L < csz))
                    def _():
                        pos = blk * L
                        idx16 = idx_s[pl.ds(pos, L)]
                        i4 = idx16 * 4
                        cx1 = plsc.load_gather(big_v, [i4])
                        cy1 = plsc.load_gather(big_v, [i4 + 1])
                        cx2 = plsc.load_gather(big_v, [i4 + 2])
                        cy2 = plsc.load_gather(big_v, [i4 + 3])
                        careas = (cx2 - cx1) * (cy2 - cy1)

                        for j in range(L):
                            cnt = cnt_smem[0]
                            ax1 = jnp.full((L,), cx1[j])
                            ay1 = jnp.full((L,), cy1[j])
                            ax2 = jnp.full((L,), cx2[j])
                            ay2 = jnp.full((L,), cy2[j])
                            aar = jnp.full((L,), careas[j])

                            def iou_blk(k, acc):
                                sl = pl.ds(k * L, L)
                                w = jnp.maximum(
                                    jnp.minimum(kx2[sl], ax2)
                                    - jnp.maximum(kx1[sl], ax1), 0.0)
                                h = jnp.maximum(
                                    jnp.minimum(ky2[sl], ay2)
                                    - jnp.maximum(ky1[sl], ay1), 0.0)
                                inter = w * h
                                denom = karea[sl] + aar - inter
                                return jnp.maximum(
                                    acc, inter - IOU_THRESHOLD * denom)

                            metric = lax.fori_loop(
                                0, NBLK, iou_blk,
                                jnp.full((L,), -1.0, jnp.float32))
                            mmax = lax.reduce_max_p.bind(metric, axes=(0,))
                            keep_j = ((cnt < MAX_DET)
                                      & (mmax <= 0.0)
                                      & (pos + j < csz))
                            m = lane0 & jnp.full((L,), keep_j)
                            idxv = jnp.full((L,), cnt, jnp.int32)
                            plsc.store_scatter(kx1, [idxv], ax1, mask=m)
                            plsc.store_scatter(ky1, [idxv], ay1, mask=m)
                            plsc.store_scatter(kx2, [idxv], ax2, mask=m)
                            plsc.store_scatter(ky2, [idxv], ay2, mask=m)
                            plsc.store_scatter(karea, [idxv], aar, mask=m)
                            oidx = jnp.full((L,), idx16[j], jnp.int32)
                            plsc.store_scatter(st_keep, [idxv], oidx, mask=m)
                            plsc.store_scatter(st_safe, [idxv], oidx, mask=m)
                            cnt_smem[0] = cnt + keep_j.astype(jnp.int32)

                    return carry

                lax.fori_loop(0, NVEC, scan_blk, jnp.int32(0))

            return rcarry

        lax.fori_loop(0, 2, round_body, jnp.int32(0))

        cnt = cnt_smem[0]
        st_cnt[...] = jnp.full((L,), cnt, jnp.int32)

        pltpu.sync_copy(cls_hbm.at[b], idx_s)
        for k in range(NBLK):
            sl = pl.ds(k * L, L)
            kidx = st_safe[sl]
            raw = st_keep[sl]
            valid = raw >= 0
            sc = plsc.load_gather(scores_v, [kidx])
            cl = plsc.load_gather(idx_s, [kidx])
            st_score[sl] = jnp.where(valid, sc, 0.0)
            st_cls[sl] = jnp.where(valid, cl, INT32_MAX)

        pltpu.sync_copy(st_keep, keep_out.at[b])
        pltpu.sync_copy(st_score, score_out.at[b])
        pltpu.sync_copy(kx1, bx1_out.at[b])
        pltpu.sync_copy(ky1, by1_out.at[b])
        pltpu.sync_copy(kx2, bx2_out.at[b])
        pltpu.sync_copy(ky2, by2_out.at[b])
        pltpu.sync_copy(st_cls, cls_out.at[b])
        pltpu.sync_copy(st_cnt, cnt_out.at[b])


_sc_nms = functools.partial(
    pl.kernel,
    out_type=(
        jax.ShapeDtypeStruct((BATCH, PAD_DET), jnp.int32),
        jax.ShapeDtypeStruct((BATCH, PAD_DET), jnp.float32),
        jax.ShapeDtypeStruct((BATCH, PAD_DET), jnp.float32),
        jax.ShapeDtypeStruct((BATCH, PAD_DET), jnp.float32),
        jax.ShapeDtypeStruct((BATCH, PAD_DET), jnp.float32),
        jax.ShapeDtypeStruct((BATCH, PAD_DET), jnp.float32),
        jax.ShapeDtypeStruct((BATCH, PAD_DET), jnp.int32),
        jax.ShapeDtypeStruct((BATCH, L), jnp.int32),
    ),
    mesh=plsc.VectorSubcoreMesh(core_axis_name="c", subcore_axis_name="s"),
    compiler_params=pltpu.CompilerParams(needs_layout_passes=False),
    scratch_types=[
        pltpu.VMEM((4 * NUM_BOXES,), jnp.float32),
        pltpu.VMEM((NUM_BOXES,), jnp.int32),
        pltpu.VMEM((NUM_BOXES,), jnp.float32),
        pltpu.VMEM((4096,), jnp.int32),
        pltpu.VMEM((PAD_DET,), jnp.float32),
        pltpu.VMEM((PAD_DET,), jnp.float32),
        pltpu.VMEM((PAD_DET,), jnp.float32),
        pltpu.VMEM((PAD_DET,), jnp.float32),
        pltpu.VMEM((PAD_DET,), jnp.float32),
        pltpu.VMEM((PAD_DET,), jnp.int32),
        pltpu.VMEM((PAD_DET,), jnp.int32),
        pltpu.VMEM((PAD_DET,), jnp.float32),
        pltpu.VMEM((PAD_DET,), jnp.int32),
        pltpu.VMEM((L,), jnp.int32),
        pltpu.SMEM((1,), jnp.int32),
        pltpu.SMEM((1,), jnp.int32),
    ],
)(_nms_body)


def kernel(scores, boxes, classes):
    B, N = scores.shape
    boxes_flat = boxes.reshape(B, 4 * N)
    keep, osc, ox1, oy1, ox2, oy2, ocl, ocnt = _sc_nms(
        scores, boxes_flat, classes.astype(jnp.int32))
    out_boxes = jnp.stack([ox1, oy1, ox2, oy2], axis=-1)[:, :MAX_DET, :]
    return (
        keep[:, :MAX_DET],
        osc[:, :MAX_DET],
        out_boxes,
        ocl[:, :MAX_DET],
        ocnt[:, 0],
    )


# speedup vs baseline: 1.9800x; 1.4902x over previous
import functools

import jax
import jax.numpy as jnp
from jax import lax
from jax.experimental import pallas as pl
from jax.experimental.pallas import tpu as pltpu
from jax.experimental.pallas import tpu_sc as plsc

IOU_THRESHOLD = 0.5
MAX_DET = 300
PAD_DET = 304
BATCH = 8
NUM_BOXES = 20000
L = 16
NBLK = PAD_DET // L
NVEC = NUM_BOXES // L
NO = 50
NI = 25
BAND = 1024
INT32_MAX = 2147483647
RB_KEY = 0
RB_IDX = NUM_BOXES
RA_KEY = 2 * NUM_BOXES
RA_IDX = 3 * NUM_BOXES


def _nms_body(scores_hbm, boxes_hbm, cls_hbm, thr_hbm,
              keep_out, score_out, bx1_out, by1_out, bx2_out, by2_out,
              cls_out, cnt_out,
              big_v, idx_s, scores_v, thr_v, hist,
              kx1, ky1, kx2, ky2, karea,
              st_keep, st_safe, st_score, st_cls, st_cnt,
              cnt_smem, c_smem):
    c = lax.axis_index("c")
    s = lax.axis_index("s")
    wid = s * 2 + c

    @pl.when(wid < BATCH)
    def _():
        b = wid
        lane = lax.iota(jnp.int32, L)
        lane0 = lane == 0
        ones = jnp.ones((L,), jnp.int32)
        zi = jnp.zeros((L,), jnp.int32)
        zf = jnp.zeros((L,), jnp.float32)
        neg1 = jnp.full((L,), -1, jnp.int32)
        m255 = jnp.full((L,), 255, jnp.int32)
        sh4 = jnp.full((L,), 4, jnp.int32)
        sh24 = jnp.full((L,), 24, jnp.int32)
        colfull = lane * jnp.int32(NVEC)

        pltpu.sync_copy(scores_hbm.at[b], scores_v)
        pltpu.sync_copy(thr_hbm.at[b], thr_v)
        cnt_smem[0] = jnp.int32(0)
        c_smem[0] = jnp.int32(0)
        msbv = jnp.full((L,), -2147483648, jnp.int32)

        def round_body(r, rcarry):
            need = (r == 0) | ((cnt_smem[0] < MAX_DET)
                               & (c_smem[0] < NUM_BOXES))

            @pl.when(need)
            def _():
                def mkkey(sf):
                    bits = plsc.bitcast(sf, jnp.int32)
                    ku = jnp.where(bits < 0, ~bits,
                                   bits ^ jnp.int32(-2147483648))
                    return ~ku

                def zero_body(i, cc):
                    hist[pl.ds(i * L, L)] = zi
                    return cc

                def pfx_body(i, carry_in):
                    vec = hist[pl.ds(i * L, L)]
                    inc = plsc.cumsum(vec)
                    hist[pl.ds(i * L, L)] = (
                        jnp.full((L,), carry_in) + inc - vec)
                    return carry_in + inc[L - 1]

                thrv = thr_v[...]
                rall = jnp.full((L,), r > 0)

                def compact_body(v, off):
                    p = v * L
                    k = mkkey(scores_v[pl.ds(p, L)])
                    iv = jnp.full((L,), p, jnp.int32) + lane
                    sel = ((k ^ msbv) <= thrv) | rall
                    plsc.store_compressed(
                        big_v.at[pl.ds(RB_KEY + off, L)],
                        plsc.bitcast(k, jnp.float32), mask=sel)
                    plsc.store_compressed(
                        big_v.at[pl.ds(RB_IDX + off, L)],
                        plsc.bitcast(iv, jnp.float32), mask=sel)
                    pc = plsc.all_reduce_population_count(sel)
                    return off + pc[0]

                csz = lax.fori_loop(0, NVEC, compact_body, jnp.int32(0))
                c_smem[0] = csz

                cpl = (csz + (L - 1)) // L
                cplv = jnp.full((L,), cpl)
                cv = jnp.full((L,), csz)
                colband = lane * cpl

                for p in range(4):
                    src_k = RB_KEY if p % 2 == 0 else RA_KEY
                    dst_k = RA_KEY if p % 2 == 0 else RB_KEY
                    shiftv = jnp.full((L,), p * 8, jnp.int32)

                    lax.fori_loop(0, 256, zero_body, jnp.int32(0))

                    def histb_outer(o, cc):
                        @pl.when(o * NI < cpl)
                        def _():
                            def histb_inner(vi, ci):
                                v = o * NI + vi
                                e = colband + v
                                m = (jnp.full((L,), v) < cplv) & (e < cv)
                                kf = plsc.load_gather(big_v, [e + src_k])
                                k = plsc.bitcast(kf, jnp.int32)
                                d = lax.shift_right_logical(k, shiftv) & m255
                                addr = lax.shift_left(d, sh4) | lane
                                plsc.addupdate_scatter(
                                    hist, [addr], ones, mask=m)
                                return ci

                            lax.fori_loop(0, NI, histb_inner, jnp.int32(0))

                        return cc

                    lax.fori_loop(0, NO, histb_outer, jnp.int32(0))
                    lax.fori_loop(0, 256, pfx_body, jnp.int32(0))

                    def scatb_outer(o, cc):
                        @pl.when(o * NI < cpl)
                        def _():
                            def scatb_inner(vi, ci):
                                v = o * NI + vi
                                e = colband + v
                                m = (jnp.full((L,), v) < cplv) & (e < cv)
                                kf = plsc.load_gather(big_v, [e + src_k])
                                inf = plsc.load_gather(
                                    big_v, [e + src_k + NUM_BOXES])
                                k = plsc.bitcast(kf, jnp.int32)
                                d = lax.shift_right_logical(k, shiftv) & m255
                                addr = lax.shift_left(d, sh4) | lane
                                ofs = plsc.load_gather(hist, [addr])
                                plsc.store_scatter(
                                    hist, [addr], ofs + 1, mask=m)
                                plsc.store_scatter(
                                    big_v, [ofs + dst_k], kf, mask=m)
                                plsc.store_scatter(
                                    big_v, [ofs + dst_k + NUM_BOXES], inf,
                                    mask=m)
                                return ci

                            lax.fori_loop(0, NI, scatb_inner, jnp.int32(0))

                        return cc

                    lax.fori_loop(0, NO, scatb_outer, jnp.int32(0))

                def save_outer(o, cc):
                    @pl.when(o * (NI * L) < csz + L)
                    def _():
                        def save_body(vi, ci):
                            p = (o * NI + vi) * L
                            f = big_v[pl.ds(RB_IDX + p, L)]
                            iv = plsc.bitcast(f, jnp.int32)
                            pos = jnp.full((L,), p, jnp.int32) + lane
                            idx_s[pl.ds(p, L)] = jnp.where(pos < cv, iv, 0)
                            return ci

                        lax.fori_loop(0, NI, save_body, jnp.int32(0))

                    return cc

                lax.fori_loop(0, NO, save_outer, jnp.int32(0))

                pltpu.sync_copy(boxes_hbm.at[b], big_v)

                for k in range(NBLK):
                    sl = pl.ds(k * L, L)
                    kx1[sl] = zf
                    ky1[sl] = zf
                    kx2[sl] = zf
                    ky2[sl] = zf
                    karea[sl] = zf
                    st_keep[sl] = neg1
                    st_safe[sl] = zi
                cnt_smem[0] = jnp.int32(0)

                def scan_blk(blk, carry):
                    @pl.when((cnt_smem[0] < MAX_DET) & (blk * L < csz))
                    def _():
                        pos = blk * L
                        idx16 = idx_s[pl.ds(pos, L)]
                        i4 = idx16 * 4
                        cx1 = plsc.load_gather(big_v, [i4])
                        cy1 = plsc.load_gather(big_v, [i4 + 1])
                        cx2 = plsc.load_gather(big_v, [i4 + 2])
                        cy2 = plsc.load_gather(big_v, [i4 + 3])
                        careas = (cx2 - cx1) * (cy2 - cy1)

                        for j in range(L):
                            cnt = cnt_smem[0]
                            ax1 = jnp.full((L,), cx1[j])
                            ay1 = jnp.full((L,), cy1[j])
                            ax2 = jnp.full((L,), cx2[j])
                            ay2 = jnp.full((L,), cy2[j])
                            aar = jnp.full((L,), careas[j])

                            def iou_blk(k, acc):
                                sl = pl.ds(k * L, L)
                                w = jnp.maximum(
                                    jnp.minimum(kx2[sl], ax2)
                                    - jnp.maximum(kx1[sl], ax1), 0.0)
                                h = jnp.maximum(
                                    jnp.minimum(ky2[sl], ay2)
                                    - jnp.maximum(ky1[sl], ay1), 0.0)
                                inter = w * h
                                denom = karea[sl] + aar - inter
                                return jnp.maximum(
                                    acc, inter - IOU_THRESHOLD * denom)

                            metric = lax.fori_loop(
                                0, NBLK, iou_blk,
                                jnp.full((L,), -1.0, jnp.float32))
                            mmax = lax.reduce_max_p.bind(metric, axes=(0,))
                            keep_j = ((cnt < MAX_DET)
                                      & (mmax <= 0.0)
                                      & (pos + j < csz))
                            m = lane0 & jnp.full((L,), keep_j)
                            idxv = jnp.full((L,), cnt, jnp.int32)
                            plsc.store_scatter(kx1, [idxv], ax1, mask=m)
                            plsc.store_scatter(ky1, [idxv], ay1, mask=m)
                            plsc.store_scatter(kx2, [idxv], ax2, mask=m)
                            plsc.store_scatter(ky2, [idxv], ay2, mask=m)
                            plsc.store_scatter(karea, [idxv], aar, mask=m)
                            oidx = jnp.full((L,), idx16[j], jnp.int32)
                            plsc.store_scatter(st_keep, [idxv], oidx, mask=m)
                            plsc.store_scatter(st_safe, [idxv], oidx, mask=m)
                            cnt_smem[0] = cnt + keep_j.astype(jnp.int32)

                    return carry

                lax.fori_loop(0, NVEC, scan_blk, jnp.int32(0))

            return rcarry

        lax.fori_loop(0, 2, round_body, jnp.int32(0))

        cnt = cnt_smem[0]
        st_cnt[...] = jnp.full((L,), cnt, jnp.int32)

        pltpu.sync_copy(cls_hbm.at[b], idx_s)
        for k in range(NBLK):
            sl = pl.ds(k * L, L)
            kidx = st_safe[sl]
            raw = st_keep[sl]
            valid = raw >= 0
            sc = plsc.load_gather(scores_v, [kidx])
            cl = plsc.load_gather(idx_s, [kidx])
            st_score[sl] = jnp.where(valid, sc, 0.0)
            st_cls[sl] = jnp.where(valid, cl, INT32_MAX)

        pltpu.sync_copy(st_keep, keep_out.at[b])
        pltpu.sync_copy(st_score, score_out.at[b])
        pltpu.sync_copy(kx1, bx1_out.at[b])
        pltpu.sync_copy(ky1, by1_out.at[b])
        pltpu.sync_copy(kx2, bx2_out.at[b])
        pltpu.sync_copy(ky2, by2_out.at[b])
        pltpu.sync_copy(st_cls, cls_out.at[b])
        pltpu.sync_copy(st_cnt, cnt_out.at[b])


_sc_nms = functools.partial(
    pl.kernel,
    out_type=(
        jax.ShapeDtypeStruct((BATCH, PAD_DET), jnp.int32),
        jax.ShapeDtypeStruct((BATCH, PAD_DET), jnp.float32),
        jax.ShapeDtypeStruct((BATCH, PAD_DET), jnp.float32),
        jax.ShapeDtypeStruct((BATCH, PAD_DET), jnp.float32),
        jax.ShapeDtypeStruct((BATCH, PAD_DET), jnp.float32),
        jax.ShapeDtypeStruct((BATCH, PAD_DET), jnp.float32),
        jax.ShapeDtypeStruct((BATCH, PAD_DET), jnp.int32),
        jax.ShapeDtypeStruct((BATCH, L), jnp.int32),
    ),
    mesh=plsc.VectorSubcoreMesh(core_axis_name="c", subcore_axis_name="s"),
    compiler_params=pltpu.CompilerParams(needs_layout_passes=False),
    scratch_types=[
        pltpu.VMEM((4 * NUM_BOXES,), jnp.float32),
        pltpu.VMEM((NUM_BOXES,), jnp.int32),
        pltpu.VMEM((NUM_BOXES,), jnp.float32),
        pltpu.VMEM((L,), jnp.int32),
        pltpu.VMEM((4096,), jnp.int32),
        pltpu.VMEM((PAD_DET,), jnp.float32),
        pltpu.VMEM((PAD_DET,), jnp.float32),
        pltpu.VMEM((PAD_DET,), jnp.float32),
        pltpu.VMEM((PAD_DET,), jnp.float32),
        pltpu.VMEM((PAD_DET,), jnp.float32),
        pltpu.VMEM((PAD_DET,), jnp.int32),
        pltpu.VMEM((PAD_DET,), jnp.int32),
        pltpu.VMEM((PAD_DET,), jnp.float32),
        pltpu.VMEM((PAD_DET,), jnp.int32),
        pltpu.VMEM((L,), jnp.int32),
        pltpu.SMEM((1,), jnp.int32),
        pltpu.SMEM((1,), jnp.int32),
    ],
)(_nms_body)


def kernel(scores, boxes, classes):
    B, N = scores.shape
    boxes_flat = boxes.reshape(B, 4 * N)
    sv, _ = lax.top_k(scores[:, ::16], 64)
    t = sv[:, 63]
    bits = lax.bitcast_convert_type(t, jnp.int32)
    ku = jnp.where(bits < 0, ~bits, bits ^ jnp.int32(-2147483648))
    thr = (~ku) ^ jnp.int32(-2147483648)
    thr16 = jnp.broadcast_to(thr[:, None], (B, L)).astype(jnp.int32)
    keep, osc, ox1, oy1, ox2, oy2, ocl, ocnt = _sc_nms(
        scores, boxes_flat, classes.astype(jnp.int32), thr16)
    out_boxes = jnp.stack([ox1, oy1, ox2, oy2], axis=-1)[:, :MAX_DET, :]
    return (
        keep[:, :MAX_DET],
        osc[:, :MAX_DET],
        out_boxes,
        ocl[:, :MAX_DET],
        ocnt[:, 0],
    )
